# Initial kernel scaffold; baseline (speedup 1.0000x reference)
import jax
import jax.numpy as jnp
from jax.experimental import pallas as pl


def kernel(features, edge_index, W_x, b_x, Ws, Wd, v, Wn, bn, Wi, bi, Wf, bf, Wo, bog, Wc, bc, ws_o, wd_o, v_o, wn_o, b_o):
    raise NotImplementedError("write your pallas kernel here")


# speedup vs baseline: 4.1068x; 4.1068x over previous
import functools

import jax
import jax.numpy as jnp
from jax import lax
from jax.experimental import pallas as pl
from jax.experimental.pallas import tpu as pltpu
from jax.experimental.pallas import tpu_sc as plsc

N = 10000
N_PAD = 10240
E = 320000
H = 128
SROWS = N_PAD // 8
NC = 2
NS = 16
NW = NC * NS
EPW = E // NW
BLK = 80
SUB = 16
NBLK = EPW // BLK
ROWS_PER_TILE = N_PAD // NS
ZCH = BLK
NCH = ROWS_PER_TILE // ZCH
SPT = SROWS // NS


def _sc_edge_body(src_hbm, dst_hbm, hs_hbm, hd_hbm, h_hbm, vw_hbm,
                  u_hbm, s_hbm,
                  u_sh, s_sh, src_v, dst_v, hs_v, hd_v, h_v, sc_v, ssc_v,
                  sidx_v, vw_v, sem):
    cid = lax.axis_index("c")
    sid = lax.axis_index("s")
    lanes = lax.iota(jnp.int32, 16)
    zero16 = jnp.zeros((16,), jnp.float32)
    onehot0 = jnp.maximum(1.0 - jnp.abs(lanes.astype(jnp.float32)), 0.0)

    def zrow(r, _):
        for c in range(H // 16):
            sc_v[r, pl.ds(c * 16, 16)] = zero16
            ssc_v[r, pl.ds(c * 16, 16)] = zero16
        return 0
    lax.fori_loop(0, ZCH, zrow, 0)
    row0 = sid * ROWS_PER_TILE
    for k in range(NCH):
        pltpu.sync_copy(sc_v, u_sh.at[pl.ds(row0 + k * ZCH, ZCH)])
    pltpu.sync_copy(ssc_v, s_sh.at[pl.ds(sid * SPT, SPT)])

    pltpu.sync_copy(vw_hbm, vw_v)
    plsc.subcore_barrier()

    ew0 = (cid * NS + sid) * EPW

    def block(b, _):
        off = ew0 + b * BLK
        pltpu.sync_copy(src_hbm.at[pl.ds(off, BLK)], src_v)
        pltpu.sync_copy(dst_hbm.at[pl.ds(off, BLK)], dst_v)

        for q in range(BLK // SUB):
            idx_s = src_v.at[pl.ds(q * SUB, SUB)]
            idx_d = dst_v.at[pl.ds(q * SUB, SUB)]
            g1 = pltpu.async_copy(hs_hbm.at[idx_s], hs_v, sem)
            g2 = pltpu.async_copy(hd_hbm.at[idx_d], hd_v, sem)
            g3 = pltpu.async_copy(h_hbm.at[idx_s], h_v, sem)
            g1.wait()
            g2.wait()
            g3.wait()

            d16 = dst_v[pl.ds(q * SUB, 16)]
            sidx_v[pl.ds(q * SUB, 16)] = lax.shift_right_logical(d16, 3)

            def edge(j, _):
                acc = zero16
                for c in range(H // 16):
                    sl = pl.ds(c * 16, 16)
                    a = hs_v[j, sl] + hd_v[j, sl]
                    ex = jnp.exp(a + a)
                    th = 1.0 - 2.0 / (ex + 1.0)
                    acc = acc + th * vw_v[sl]
                for k in (1, 2, 4, 8):
                    acc = acc + acc.at[jnp.bitwise_xor(lanes, k)].get(
                        mode="promise_in_bounds")
                e = jnp.exp(acc)
                r = q * SUB + j
                for c in range(H // 16):
                    sl = pl.ds(c * 16, 16)
                    sc_v[r, sl] = e * h_v[j, sl]
                dsp = d16.at[jnp.full((16,), j, jnp.int32)].get(
                    mode="promise_in_bounds")
                m = jnp.bitwise_and(dsp, 7)
                eoh = e * onehot0
                for c in range(H // 16):
                    df = jnp.abs(m - c).astype(jnp.float32)
                    ind = jnp.maximum(1.0 - df, 0.0)
                    ssc_v[r, pl.ds(c * 16, 16)] = eoh * ind
                return 0
            lax.fori_loop(0, SUB, edge, 0)

        pltpu.sync_copy(sc_v, u_sh.at[dst_v], add=True)
        pltpu.sync_copy(ssc_v, s_sh.at[sidx_v], add=True)
        return 0

    lax.fori_loop(0, NBLK, block, 0)
    plsc.subcore_barrier()

    for k in range(NCH):
        sl = pl.ds(row0 + k * ZCH, ZCH)
        pltpu.sync_copy(u_sh.at[sl], u_hbm.at[cid].at[sl])
    sls = pl.ds(sid * SPT, SPT)
    pltpu.sync_copy(s_sh.at[sls], s_hbm.at[cid].at[sls])
    return None


def _make_sc_edge():
    mesh = plsc.VectorSubcoreMesh(core_axis_name="c", subcore_axis_name="s")
    return functools.partial(
        pl.kernel,
        mesh=mesh,
        out_type=[jax.ShapeDtypeStruct((NC, N_PAD, H), jnp.float32),
                  jax.ShapeDtypeStruct((NC, SROWS, H), jnp.float32)],
        scratch_types=[
            pltpu.VMEM_SHARED((N_PAD, H), jnp.float32),
            pltpu.VMEM_SHARED((SROWS, H), jnp.float32),
            pltpu.VMEM((BLK,), jnp.int32),
            pltpu.VMEM((BLK,), jnp.int32),
            pltpu.VMEM((SUB, H), jnp.float32),
            pltpu.VMEM((SUB, H), jnp.float32),
            pltpu.VMEM((SUB, H), jnp.float32),
            pltpu.VMEM((BLK, H), jnp.float32),
            pltpu.VMEM((BLK, H), jnp.float32),
            pltpu.VMEM((BLK,), jnp.int32),
            pltpu.VMEM((H,), jnp.float32),
            pltpu.SemaphoreType.DMA,
        ],
    )(_sc_edge_body)


_sc_edge = _make_sc_edge()


_RB = 1000
_GRID = N // _RB


def _full(shape):
    return pl.BlockSpec(shape, lambda i: tuple(0 for _ in shape))


def _rows(width=H):
    return pl.BlockSpec((_RB, width), lambda i: (i, 0))


def _u_spec():
    return pl.BlockSpec((NC, _RB, H), lambda i: (0, i, 0))


def _s_spec():
    return pl.BlockSpec((NC, _RB, 16), lambda i: (0, i, 0))


def _pre_body(x_ref, wx_ref, bx_ref, ws_ref, wd_ref, h_ref, hs_ref, hd_ref):
    h = jnp.tanh(
        jnp.dot(x_ref[...], wx_ref[...], preferred_element_type=jnp.float32)
        + bx_ref[...])
    h_ref[...] = h
    hs_ref[...] = jnp.dot(h, ws_ref[...], preferred_element_type=jnp.float32)
    hd_ref[...] = jnp.dot(h, wd_ref[...], preferred_element_type=jnp.float32)


def _tc_pre(x, wx, bx, ws, wd):
    return pl.pallas_call(
        _pre_body,
        grid=(_GRID,),
        in_specs=[_rows(), _full((H, H)), _full((1, H)), _full((H, H)),
                  _full((H, H))],
        out_specs=[_rows(), _rows(), _rows()],
        out_shape=[jax.ShapeDtypeStruct((N, H), jnp.float32)] * 3,
    )(x, wx, bx, ws, wd)


def _agg_from_u(u_ref, s_ref):
    u = u_ref[0] + u_ref[1]
    s = s_ref[0, :, 0:1] + s_ref[1, :, 0:1]
    return jnp.where(s > 0.0, u / s, 0.0)


def _mid_body(u_ref, s_ref, h_ref, c_ref, wn_ref, bn_ref, wia_ref, wib_ref, bi_ref,
              wfa_ref, wfb_ref, bf_ref, woa_ref, wob_ref, bo_ref,
              wca_ref, wcb_ref, bc_ref, wsn_ref, wdn_ref,
              ho_ref, co_ref, hs_ref, hd_ref):
    dot = lambda a, b: jnp.dot(a, b, preferred_element_type=jnp.float32)
    agg = _agg_from_u(u_ref, s_ref)
    h_tmp = jnp.tanh(dot(agg, wn_ref[...]) + bn_ref[...])
    h = h_ref[...]
    ig = jax.nn.sigmoid(dot(h, wia_ref[...]) + dot(h_tmp, wib_ref[...])
                        + bi_ref[...])
    fg = jax.nn.sigmoid(dot(h, wfa_ref[...]) + dot(h_tmp, wfb_ref[...])
                        + bf_ref[...])
    og = jax.nn.sigmoid(dot(h, woa_ref[...]) + dot(h_tmp, wob_ref[...])
                        + bo_ref[...])
    cg = jnp.tanh(dot(h, wca_ref[...]) + dot(h_tmp, wcb_ref[...])
                  + bc_ref[...])
    c_new = fg * c_ref[...] + ig * cg
    h_new = og * jnp.tanh(c_new)
    ho_ref[...] = h_new
    co_ref[...] = c_new
    hs_ref[...] = dot(h_new, wsn_ref[...])
    hd_ref[...] = dot(h_new, wdn_ref[...])


def _tc_mid(u, sv, h, c, wn, bn, wia, wib, bi, wfa, wfb, bf, woa, wob, bo,
            wca, wcb, bc, wsn, wdn):
    w = _full((H, H))
    b = _full((1, H))
    return pl.pallas_call(
        _mid_body,
        grid=(_GRID,),
        in_specs=[_u_spec(), _s_spec(), _rows(), _rows(),
                  w, b, w, w, b, w, w, b, w, w, b, w, w, b, w, w],
        out_specs=[_rows(), _rows(), _rows(), _rows()],
        out_shape=[jax.ShapeDtypeStruct((N, H), jnp.float32)] * 4,
    )(u, sv, h, c, wn, bn, wia, wib, bi, wfa, wfb, bf, woa, wob, bo,
      wca, wcb, bc, wsn, wdn)


def _fin_body(u_ref, s_ref, wo_ref, bo_ref, out_ref):
    agg = _agg_from_u(u_ref, s_ref)
    out_ref[...] = (jnp.dot(agg, wo_ref[...], preferred_element_type=jnp.float32)
                    + bo_ref[...])


def _tc_fin(u, sv, wo_pad, bo_pad):
    return pl.pallas_call(
        _fin_body,
        grid=(_GRID,),
        in_specs=[_u_spec(), _s_spec(), _full((H, H)), _full((1, H))],
        out_specs=_rows(),
        out_shape=jax.ShapeDtypeStruct((N, H), jnp.float32),
    )(u, sv, wo_pad, bo_pad)


def _s_view(s_out):
    return s_out.reshape(NC, N_PAD, 16)[:, :N]


def kernel(features, edge_index, W_x, b_x, Ws, Wd, v, Wn, bn, Wi, bi,
           Wf, bf, Wo, bog, Wc, bc, ws_o, wd_o, v_o, wn_o, b_o):
    src = edge_index[0].astype(jnp.int32)
    dst = edge_index[1].astype(jnp.int32)
    L = wn_o.shape[1]

    h, hs, hd = _tc_pre(features, W_x, b_x.reshape(1, H), Ws[0], Wd[0])
    c = jnp.zeros((N, H), jnp.float32)

    for i in range(2):
        u, s_out = _sc_edge(src, dst, hs, hd, h, v[i])
        wsn, wdn = (Ws[1], Wd[1]) if i == 0 else (ws_o, wd_o)
        h, c, hs, hd = _tc_mid(
            u, _s_view(s_out), h, c, Wn[i], bn[i].reshape(1, H),
            Wi[i][:H], Wi[i][H:], bi[i].reshape(1, H),
            Wf[i][:H], Wf[i][H:], bf[i].reshape(1, H),
            Wo[i][:H], Wo[i][H:], bog[i].reshape(1, H),
            Wc[i][:H], Wc[i][H:], bc[i].reshape(1, H),
            wsn, wdn)

    u, s_out = _sc_edge(src, dst, hs, hd, h, v_o)
    wo_pad = jnp.zeros((H, H), jnp.float32).at[:, :L].set(wn_o)
    bo_pad = jnp.zeros((1, H), jnp.float32).at[0, :L].set(b_o)
    out = _tc_fin(u, _s_view(s_out), wo_pad, bo_pad)
    return out[:, :L]
